# baseline (speedup 1.0000x reference)
import functools

import jax
import jax.numpy as jnp
from jax.experimental import pallas as pl
from jax.experimental.pallas import tpu as pltpu


def _copy_scatter_kernel(idx_ref, x_ref, copy_ref, out_ref, *, block_rows):
    i = pl.program_id(0)
    out_ref[...] = x_ref[...]
    idx = idx_ref[0]
    base = i * block_rows
    @pl.when((idx >= base) & (idx < base + block_rows))
    def _():
        out_ref[pl.ds(idx - base, 1), :] = copy_ref[...]


def _pick_block_rows(rows):
    for b in (8192, 10000, 8000, 5000, 4000, 2000, 1000, 500, 200, 100, 8):
        if rows % b == 0:
            return b
    return rows


def kernel(x, copy_tensor, indices):
    rows, cols = x.shape
    block_rows = _pick_block_rows(rows)
    grid = rows // block_rows
    return pl.pallas_call(
        functools.partial(_copy_scatter_kernel, block_rows=block_rows),
        grid_spec=pltpu.PrefetchScalarGridSpec(
            num_scalar_prefetch=1,
            grid=(grid,),
            in_specs=[
                pl.BlockSpec((block_rows, cols), lambda i, idx: (i, 0)),
                pl.BlockSpec((1, cols), lambda i, idx: (0, 0)),
            ],
            out_specs=pl.BlockSpec((block_rows, cols), lambda i, idx: (i, 0)),
        ),
        out_shape=jax.ShapeDtypeStruct((rows, cols), x.dtype),
    )(indices, x, copy_tensor)
